# baseline (speedup 1.0000x reference)
import functools

import jax
import jax.numpy as jnp
from jax import lax
from jax.experimental import pallas as pl
from jax.experimental.pallas import tpu as pltpu
from jax.experimental.pallas import tpu_sc as plsc

_L = 128
_H = 1024
_O = 24576
_B = 2048

_B_SC = 512
_B_TC = _B - _B_SC
_M = _L + _B_SC

_BN = 1536
_NT = _O // _BN

_NC, _NS = 2, 16
_NW = _NC * _NS
_RPW = 16
_NW_ACT = _B_SC // _RPW


def _sc_gather_body(emb, idx2, out, idx_v, buf, sem):
    wid = lax.axis_index("s") * _NC + lax.axis_index("c")

    @pl.when(wid < _NW_ACT)
    def _():
        pltpu.sync_copy(idx2.at[pl.ds(wid, 1)], idx_v)
        pltpu.async_copy(emb.at[idx_v.at[0]], buf, sem).wait()
        pltpu.sync_copy(buf, out.at[pl.ds(wid * _RPW, _RPW)])


@functools.cache
def _sc_gather():
    return pl.kernel(
        _sc_gather_body,
        out_type=jax.ShapeDtypeStruct((_B_SC, _H), jnp.float32),
        mesh=plsc.VectorSubcoreMesh(
            core_axis_name="c", subcore_axis_name="s", num_cores=_NC
        ),
        scratch_types=[
            pltpu.VMEM((1, _RPW), jnp.int32),
            pltpu.VMEM((_RPW, _H), jnp.float32),
            pltpu.SemaphoreType.DMA,
        ],
    )


def _mlp_body(idx_tc, emb, emb_sel, w1, b1, w2, b2, out, h1, oh):
    @pl.when(pl.program_id(0) == 0)
    def _():
        rows = jnp.concatenate([emb[...], emb_sel[...]], axis=0)
        h1[...] = jnp.tanh(
            jnp.dot(rows, w1[...], preferred_element_type=jnp.float32)
            + b1[...]
        )
        cols = lax.broadcasted_iota(jnp.int32, (_B_TC, _L), 1)
        oh[...] = jnp.where(cols == idx_tc[...], 1.0, 0.0).astype(jnp.float32)

    h2 = (
        jnp.dot(h1[...], w2[...], preferred_element_type=jnp.float32)
        + b2[...]
    )
    out[...] = jnp.concatenate(
        [
            jnp.dot(
                oh[...].astype(jnp.bfloat16),
                h2[:_L].astype(jnp.bfloat16),
                preferred_element_type=jnp.float32,
            ),
            h2[_L:],
        ],
        axis=0,
    )


def _table_mlp_expand(idx_tc, emb_table, emb_sel, W1, b1, W2, b2):
    return pl.pallas_call(
        _mlp_body,
        grid=(_NT,),
        in_specs=[
            pl.BlockSpec((_B_TC, 1), lambda j: (0, 0)),
            pl.BlockSpec((_L, _H), lambda j: (0, 0)),
            pl.BlockSpec((_B_SC, _H), lambda j: (0, 0)),
            pl.BlockSpec((_H, _H), lambda j: (0, 0)),
            pl.BlockSpec((1, _H), lambda j: (0, 0)),
            pl.BlockSpec((_H, _BN), lambda j: (0, j)),
            pl.BlockSpec((1, _BN), lambda j: (0, j)),
        ],
        out_specs=pl.BlockSpec((_B, _BN), lambda j: (0, j)),
        out_shape=jax.ShapeDtypeStruct((_B, _O), jnp.float32),
        scratch_shapes=[
            pltpu.VMEM((_M, _H), jnp.float32),
            pltpu.VMEM((_B_TC, _L), jnp.float32),
        ],
    )(idx_tc, emb_table, emb_sel, W1, b1.reshape(1, _H), W2, b2.reshape(1, _O))


def kernel(prefix, emb_table, W1, b1, W2, b2):
    flat = prefix.astype(jnp.int32).reshape(_B)
    idx_tc = flat[:_B_TC].reshape(_B_TC, 1)
    idx_sc = flat[_B_TC:].reshape(_NW_ACT, _RPW)
    emb_sel = _sc_gather()(emb_table, idx_sc)
    out = _table_mlp_expand(idx_tc, emb_table, emb_sel, W1, b1, W2, b2)
    return out.reshape(prefix.shape[0], prefix.shape[1], _O)


# speedup vs baseline: 1.0279x; 1.0279x over previous
import functools

import jax
import jax.numpy as jnp
from jax import lax
from jax.experimental import pallas as pl
from jax.experimental.pallas import tpu as pltpu
from jax.experimental.pallas import tpu_sc as plsc

_L = 128
_H = 1024
_O = 24576
_B = 2048

_B_SC = 256
_B_TC = _B - _B_SC
_M = _L + _B_SC

_BN = 1536
_NT = _O // _BN

_NC, _NS = 2, 16
_NW = _NC * _NS
_RPW = 8
_NW_ACT = _B_SC // _RPW


def _sc_gather_body(emb, idx2, out, idx_v, buf, sem):
    wid = lax.axis_index("s") * _NC + lax.axis_index("c")

    @pl.when(wid < _NW_ACT)
    def _():
        pltpu.sync_copy(idx2.at[pl.ds(wid, 1)], idx_v)
        pltpu.async_copy(emb.at[idx_v.at[0]], buf, sem).wait()
        pltpu.sync_copy(buf, out.at[pl.ds(wid * _RPW, _RPW)])


@functools.cache
def _sc_gather():
    return pl.kernel(
        _sc_gather_body,
        out_type=jax.ShapeDtypeStruct((_B_SC, _H), jnp.float32),
        mesh=plsc.VectorSubcoreMesh(
            core_axis_name="c", subcore_axis_name="s", num_cores=_NC
        ),
        scratch_types=[
            pltpu.VMEM((1, _RPW), jnp.int32),
            pltpu.VMEM((_RPW, _H), jnp.float32),
            pltpu.SemaphoreType.DMA,
        ],
    )


def _mlp_body(idx_tc, emb, emb_sel, w1, b1, w2, b2, out, h1, oh):
    @pl.when(pl.program_id(0) == 0)
    def _():
        rows = jnp.concatenate([emb[...], emb_sel[...]], axis=0)
        h1[...] = jnp.tanh(
            jnp.dot(rows, w1[...], preferred_element_type=jnp.float32)
            + b1[...]
        )
        cols = lax.broadcasted_iota(jnp.int32, (_B_TC, _L), 1)
        oh[...] = jnp.where(cols == idx_tc[...], 1.0, 0.0).astype(jnp.float32)

    h2 = (
        jnp.dot(h1[...], w2[...], preferred_element_type=jnp.float32)
        + b2[...]
    )
    out[...] = jnp.concatenate(
        [
            jnp.dot(oh[...], h2[:_L], preferred_element_type=jnp.float32),
            h2[_L:],
        ],
        axis=0,
    )


def _table_mlp_expand(idx_tc, emb_table, emb_sel, W1, b1, W2, b2):
    return pl.pallas_call(
        _mlp_body,
        grid=(_NT,),
        in_specs=[
            pl.BlockSpec((_B_TC, 1), lambda j: (0, 0)),
            pl.BlockSpec((_L, _H), lambda j: (0, 0)),
            pl.BlockSpec((_B_SC, _H), lambda j: (0, 0)),
            pl.BlockSpec((_H, _H), lambda j: (0, 0)),
            pl.BlockSpec((1, _H), lambda j: (0, 0)),
            pl.BlockSpec((_H, _BN), lambda j: (0, j)),
            pl.BlockSpec((1, _BN), lambda j: (0, j)),
        ],
        out_specs=pl.BlockSpec((_B, _BN), lambda j: (0, j)),
        out_shape=jax.ShapeDtypeStruct((_B, _O), jnp.float32),
        scratch_shapes=[
            pltpu.VMEM((_M, _H), jnp.float32),
            pltpu.VMEM((_B_TC, _L), jnp.float32),
        ],
    )(idx_tc, emb_table, emb_sel, W1, b1.reshape(1, _H), W2, b2.reshape(1, _O))


def kernel(prefix, emb_table, W1, b1, W2, b2):
    flat = prefix.astype(jnp.int32).reshape(_B)
    idx_tc = flat[:_B_TC].reshape(_B_TC, 1)
    idx_sc = flat[_B_TC:].reshape(_NW_ACT, _RPW)
    emb_sel = _sc_gather()(emb_table, idx_sc)
    out = _table_mlp_expand(idx_tc, emb_table, emb_sel, W1, b1, W2, b2)
    return out.reshape(prefix.shape[0], prefix.shape[1], _O)


# speedup vs baseline: 1.0280x; 1.0001x over previous
import functools

import jax
import jax.numpy as jnp
from jax import lax
from jax.experimental import pallas as pl
from jax.experimental.pallas import tpu as pltpu
from jax.experimental.pallas import tpu_sc as plsc

_L = 128
_H = 1024
_O = 24576
_B = 2048

_B_SC = 256
_B_TC = _B - _B_SC
_M = _L + _B_SC

_BN = 2048
_NT = _O // _BN

_NC, _NS = 2, 16
_NW = _NC * _NS
_RPW = 8
_NW_ACT = _B_SC // _RPW


def _sc_gather_body(emb, idx2, out, idx_v, buf, sem):
    wid = lax.axis_index("s") * _NC + lax.axis_index("c")

    @pl.when(wid < _NW_ACT)
    def _():
        pltpu.sync_copy(idx2.at[pl.ds(wid, 1)], idx_v)
        pltpu.async_copy(emb.at[idx_v.at[0]], buf, sem).wait()
        pltpu.sync_copy(buf, out.at[pl.ds(wid * _RPW, _RPW)])


@functools.cache
def _sc_gather():
    return pl.kernel(
        _sc_gather_body,
        out_type=jax.ShapeDtypeStruct((_B_SC, _H), jnp.float32),
        mesh=plsc.VectorSubcoreMesh(
            core_axis_name="c", subcore_axis_name="s", num_cores=_NC
        ),
        scratch_types=[
            pltpu.VMEM((1, _RPW), jnp.int32),
            pltpu.VMEM((_RPW, _H), jnp.float32),
            pltpu.SemaphoreType.DMA,
        ],
    )


def _mlp_body(idx_tc, emb, emb_sel, w1, b1, w2, b2, out, h1, oh):
    @pl.when(pl.program_id(0) == 0)
    def _():
        rows = jnp.concatenate([emb[...], emb_sel[...]], axis=0)
        h1[...] = jnp.tanh(
            jnp.dot(rows, w1[...], preferred_element_type=jnp.float32)
            + b1[...]
        )
        cols = lax.broadcasted_iota(jnp.int32, (_B_TC, _L), 1)
        oh[...] = jnp.where(cols == idx_tc[...], 1.0, 0.0).astype(jnp.float32)

    h2 = (
        jnp.dot(h1[...], w2[...], preferred_element_type=jnp.float32)
        + b2[...]
    )
    out[...] = jnp.concatenate(
        [
            jnp.dot(oh[...], h2[:_L], preferred_element_type=jnp.float32),
            h2[_L:],
        ],
        axis=0,
    )


def _table_mlp_expand(idx_tc, emb_table, emb_sel, W1, b1, W2, b2):
    return pl.pallas_call(
        _mlp_body,
        grid=(_NT,),
        in_specs=[
            pl.BlockSpec((_B_TC, 1), lambda j: (0, 0)),
            pl.BlockSpec((_L, _H), lambda j: (0, 0)),
            pl.BlockSpec((_B_SC, _H), lambda j: (0, 0)),
            pl.BlockSpec((_H, _H), lambda j: (0, 0)),
            pl.BlockSpec((1, _H), lambda j: (0, 0)),
            pl.BlockSpec((_H, _BN), lambda j: (0, j)),
            pl.BlockSpec((1, _BN), lambda j: (0, j)),
        ],
        out_specs=pl.BlockSpec((_B, _BN), lambda j: (0, j)),
        out_shape=jax.ShapeDtypeStruct((_B, _O), jnp.float32),
        scratch_shapes=[
            pltpu.VMEM((_M, _H), jnp.float32),
            pltpu.VMEM((_B_TC, _L), jnp.float32),
        ],
    )(idx_tc, emb_table, emb_sel, W1, b1.reshape(1, _H), W2, b2.reshape(1, _O))


def kernel(prefix, emb_table, W1, b1, W2, b2):
    flat = prefix.astype(jnp.int32).reshape(_B)
    idx_tc = flat[:_B_TC].reshape(_B_TC, 1)
    idx_sc = flat[_B_TC:].reshape(_NW_ACT, _RPW)
    emb_sel = _sc_gather()(emb_table, idx_sc)
    out = _table_mlp_expand(idx_tc, emb_table, emb_sel, W1, b1, W2, b2)
    return out.reshape(prefix.shape[0], prefix.shape[1], _O)
